# speedup vs baseline: 5.9364x; 5.9364x over previous
import functools

import jax
import jax.numpy as jnp
from jax import lax
from jax.experimental import pallas as pl
from jax.experimental.pallas import tpu as pltpu
from jax.experimental.pallas import tpu_sc as plsc

NUM_PHONEMES = 80
NUM_STRESS = 4
HIDDEN = 256

NC = 2
NS = 16
NW = NC * NS
LANES = 16

CHUNK = 128


def _combine_body(p_ref, s_ref, o_ref):
    o_ref[...] = p_ref[...][:, None, :] + s_ref[...][None, :, :]


def _combine_tables(phoneme_table, stress_table):
    out3 = pl.pallas_call(
        _combine_body,
        out_shape=jax.ShapeDtypeStruct((NUM_PHONEMES, NUM_STRESS, HIDDEN), jnp.float32),
    )(phoneme_table, stress_table)
    return out3.reshape(NUM_PHONEMES * NUM_STRESS, HIDDEN)


def _make_sc_gather(n_tokens):
    assert n_tokens % (NW * CHUNK) == 0
    b_per_w = n_tokens // NW
    n_chunks = b_per_w // CHUNK
    mesh = plsc.VectorSubcoreMesh(core_axis_name="c", subcore_axis_name="s")

    @functools.partial(
        pl.kernel,
        mesh=mesh,
        out_type=jax.ShapeDtypeStruct((n_tokens, HIDDEN), jnp.float32),
        scratch_types=[
            pltpu.VMEM((CHUNK,), jnp.int32),
            pltpu.VMEM((CHUNK,), jnp.int32),
            pltpu.VMEM((CHUNK,), jnp.int32),
            pltpu.VMEM((CHUNK, HIDDEN), jnp.float32),
            pltpu.SemaphoreType.DMA,
        ],
    )
    def sc_gather(tbl_hbm, pid_hbm, sid_hbm, out_hbm, pidv, sidv, idxv, rowsv, sem):
        wid = lax.axis_index("s") * NC + lax.axis_index("c")
        base = wid * b_per_w

        def chunk_body(i, carry):
            off = base + i * CHUNK
            pltpu.sync_copy(pid_hbm.at[pl.ds(off, CHUNK)], pidv)
            pltpu.sync_copy(sid_hbm.at[pl.ds(off, CHUNK)], sidv)
            for j in range(CHUNK // LANES):
                sl = pl.ds(j * LANES, LANES)
                idxv[sl] = pidv[sl] * NUM_STRESS + sidv[sl]
            pltpu.async_copy(tbl_hbm.at[idxv], rowsv, sem).wait()
            pltpu.sync_copy(rowsv, out_hbm.at[pl.ds(off, CHUNK)])
            return carry

        lax.fori_loop(0, n_chunks, chunk_body, 0)

    return sc_gather


def kernel(phoneme_ids, stress_ids, phoneme_table, stress_table):
    B, L = phoneme_ids.shape
    n_tokens = B * L
    combined = _combine_tables(phoneme_table.astype(jnp.float32),
                               stress_table.astype(jnp.float32))
    pid = phoneme_ids.reshape(-1).astype(jnp.int32)
    sid = stress_ids.reshape(-1).astype(jnp.int32)
    out = _make_sc_gather(n_tokens)(combined, pid, sid)
    return out.reshape(B, L, HIDDEN)


# speedup vs baseline: 5.9630x; 1.0045x over previous
import functools

import jax
import jax.numpy as jnp
from jax import lax
from jax.experimental import pallas as pl
from jax.experimental.pallas import tpu as pltpu
from jax.experimental.pallas import tpu_sc as plsc

NUM_PHONEMES = 80
NUM_STRESS = 4
HIDDEN = 256

NC = 2
NS = 16
NW = NC * NS
LANES = 16

CHUNK = 128


def _combine_body(p_ref, s_ref, o_ref):
    o_ref[...] = p_ref[...][:, None, :] + s_ref[...][None, :, :]


def _combine_tables(phoneme_table, stress_table):
    out3 = pl.pallas_call(
        _combine_body,
        out_shape=jax.ShapeDtypeStruct((NUM_PHONEMES, NUM_STRESS, HIDDEN), jnp.float32),
    )(phoneme_table, stress_table)
    return out3.reshape(NUM_PHONEMES * NUM_STRESS, HIDDEN)


def _make_sc_gather(n_tokens):
    assert n_tokens % (NW * CHUNK) == 0
    b_per_w = n_tokens // NW
    n_chunks = b_per_w // CHUNK
    assert n_chunks % 2 == 0 and n_chunks >= 4
    n_pairs = n_chunks // 2
    mesh = plsc.VectorSubcoreMesh(core_axis_name="c", subcore_axis_name="s")

    @functools.partial(
        pl.kernel,
        mesh=mesh,
        out_type=jax.ShapeDtypeStruct((n_tokens, HIDDEN), jnp.float32),
        scratch_types=[
            pltpu.VMEM((2, CHUNK), jnp.int32),
            pltpu.VMEM((2, CHUNK), jnp.int32),
            pltpu.VMEM((2, CHUNK), jnp.int32),
            pltpu.VMEM((2, CHUNK, HIDDEN), jnp.float32),
            pltpu.SemaphoreType.DMA,
            pltpu.SemaphoreType.DMA,
            pltpu.SemaphoreType.DMA,
            pltpu.SemaphoreType.DMA,
        ],
    )
    def sc_gather(tbl_hbm, pid_hbm, sid_hbm, out_hbm, pidv, sidv, idxv, rowsv,
                  g0, g1, o0, o1):
        gsem = (g0, g1)
        osem = (o0, o1)
        wid = lax.axis_index("s") * NC + lax.axis_index("c")
        base = wid * b_per_w

        def start_chunk(i, b):
            off = base + i * CHUNK
            pltpu.sync_copy(pid_hbm.at[pl.ds(off, CHUNK)], pidv.at[b])
            pltpu.sync_copy(sid_hbm.at[pl.ds(off, CHUNK)], sidv.at[b])
            for j in range(CHUNK // LANES):
                sl = pl.ds(j * LANES, LANES)
                idxv.at[b][sl] = pidv.at[b][sl] * NUM_STRESS + sidv.at[b][sl]
            pltpu.async_copy(tbl_hbm.at[idxv.at[b]], rowsv.at[b], gsem[b])

        def finish_chunk(i, b):
            off = base + i * CHUNK
            pltpu.make_async_copy(tbl_hbm.at[idxv.at[b]], rowsv.at[b],
                                  gsem[b]).wait()
            pltpu.async_copy(rowsv.at[b], out_hbm.at[pl.ds(off, CHUNK)], osem[b])

        def drain_out(i, b):
            pltpu.make_async_copy(rowsv.at[b],
                                  out_hbm.at[pl.ds(base + i * CHUNK, CHUNK)],
                                  osem[b]).wait()

        start_chunk(0, 0)
        start_chunk(1, 1)
        finish_chunk(0, 0)

        def pair_body(gg, carry):
            i0 = 2 * gg
            drain_out(i0 - 2, 0)
            start_chunk(i0, 0)
            finish_chunk(i0 - 1, 1)
            drain_out(i0 - 1, 1)
            start_chunk(i0 + 1, 1)
            finish_chunk(i0, 0)
            return carry

        lax.fori_loop(1, n_pairs, pair_body, 0)

        finish_chunk(n_chunks - 1, 1)
        drain_out(n_chunks - 2, 0)
        drain_out(n_chunks - 1, 1)

    return sc_gather


def kernel(phoneme_ids, stress_ids, phoneme_table, stress_table):
    B, L = phoneme_ids.shape
    n_tokens = B * L
    combined = _combine_tables(phoneme_table.astype(jnp.float32),
                               stress_table.astype(jnp.float32))
    pid = phoneme_ids.reshape(-1).astype(jnp.int32)
    sid = stress_ids.reshape(-1).astype(jnp.int32)
    out = _make_sc_gather(n_tokens)(combined, pid, sid)
    return out.reshape(B, L, HIDDEN)
